# baseline (speedup 1.0000x reference)
import functools

import jax
import jax.numpy as jnp
from jax import lax
from jax.experimental import pallas as pl
from jax.experimental.pallas import tpu as pltpu
from jax.experimental.pallas import tpu_sc as plsc

NC = 2
NS = 16
NW = NC * NS


def _make_lookup(B, D, CH):
    b_per_w = B // NW
    n_ch = b_per_w // CH
    mesh = plsc.VectorSubcoreMesh(core_axis_name="c", subcore_axis_name="s")

    @functools.partial(
        pl.kernel,
        mesh=mesh,
        out_type=jax.ShapeDtypeStruct((B, D), jnp.float32),
        scratch_types=[
            pltpu.VMEM((CH,), jnp.int32),
            pltpu.VMEM((CH, D), jnp.float32),
            pltpu.SemaphoreType.DMA,
        ],
        compiler_params=pltpu.CompilerParams(use_tc_tiling_on_sc=False),
    )
    def lookup(idx_hbm, table_hbm, out_hbm, idx_v, rows_v, sem):
        wid = lax.axis_index("s") * NC + lax.axis_index("c")
        base = wid * b_per_w

        def body(i, carry):
            off = base + i * CH
            pltpu.sync_copy(idx_hbm.at[pl.ds(off, CH)], idx_v)
            pltpu.async_copy(table_hbm.at[idx_v], rows_v, sem).wait()
            pltpu.sync_copy(rows_v, out_hbm.at[pl.ds(off, CH)])
            return carry

        lax.fori_loop(0, n_ch, body, 0)

    return lookup


def kernel(x, table):
    B0, B1 = x.shape
    V, D = table.shape
    B = B0 * B1
    xf = x.reshape(B).astype(jnp.int32)
    out = _make_lookup(B, D, CH=512)(xf, table)
    return out.reshape(B0, B1, D)


# speedup vs baseline: 1.0434x; 1.0434x over previous
import functools

import jax
import jax.numpy as jnp
from jax import lax
from jax.experimental import pallas as pl
from jax.experimental.pallas import tpu as pltpu
from jax.experimental.pallas import tpu_sc as plsc

NC = 2
NS = 16
NW = NC * NS


def _make_lookup(B, D, CH, NBUF):
    b_per_w = B // NW
    n_ch = b_per_w // CH
    assert n_ch % NBUF == 0 and n_ch >= NBUF
    mesh = plsc.VectorSubcoreMesh(core_axis_name="c", subcore_axis_name="s")

    @functools.partial(
        pl.kernel,
        mesh=mesh,
        out_type=jax.ShapeDtypeStruct((B, D), jnp.float32),
        scratch_types=[
            pltpu.VMEM((b_per_w,), jnp.int32),
            [pltpu.VMEM((CH, D), jnp.float32) for _ in range(NBUF)],
            [pltpu.SemaphoreType.DMA for _ in range(NBUF)],
            [pltpu.SemaphoreType.DMA for _ in range(NBUF)],
        ],
        compiler_params=pltpu.CompilerParams(use_tc_tiling_on_sc=False),
    )
    def lookup(idx_hbm, table_hbm, out_hbm, idx_v, rows, g_sems, w_sems):
        wid = lax.axis_index("s") * NC + lax.axis_index("c")
        base = wid * b_per_w
        pltpu.sync_copy(idx_hbm.at[pl.ds(base, b_per_w)], idx_v)

        def gather_start(i, b):
            pltpu.async_copy(
                table_hbm.at[idx_v.at[pl.ds(i * CH, CH)]], rows[b], g_sems[b]
            )

        def writeback_start(i, b):
            pltpu.async_copy(
                rows[b], out_hbm.at[pl.ds(base + i * CH, CH)], w_sems[b]
            )

        for b in range(NBUF):
            gather_start(b, b)

        def body(g, carry):
            for b in range(NBUF):
                i = g + b
                pltpu.make_async_copy(
                    table_hbm.at[idx_v.at[pl.ds(0, CH)]], rows[b], g_sems[b]
                ).wait()
                writeback_start(i, b)

                @pl.when(i + NBUF < n_ch)
                def _():
                    pltpu.make_async_copy(
                        rows[b], out_hbm.at[pl.ds(base, CH)], w_sems[b]
                    ).wait()
                    gather_start(i + NBUF, b)

            return carry

        lax.fori_loop(0, n_ch // NBUF, lambda t, c: body(t * NBUF, c), 0,
                      unroll=False)

        for b in range(NBUF):
            pltpu.make_async_copy(
                rows[b], out_hbm.at[pl.ds(base, CH)], w_sems[b]
            ).wait()

    return lookup


def kernel(x, table):
    B0, B1 = x.shape
    V, D = table.shape
    B = B0 * B1
    xf = x.reshape(B).astype(jnp.int32)
    out = _make_lookup(B, D, CH=320, NBUF=4)(xf, table)
    return out.reshape(B0, B1, D)


# speedup vs baseline: 1.0446x; 1.0011x over previous
import functools

import jax
import jax.numpy as jnp
from jax import lax
from jax.experimental import pallas as pl
from jax.experimental.pallas import tpu as pltpu
from jax.experimental.pallas import tpu_sc as plsc

NC = 2
NS = 16
NW = NC * NS


def _make_lookup(R, C, D, NBUF):
    rows_per_w = R // NW
    assert rows_per_w % NBUF == 0
    mesh = plsc.VectorSubcoreMesh(core_axis_name="c", subcore_axis_name="s")

    @functools.partial(
        pl.kernel,
        mesh=mesh,
        out_type=jax.ShapeDtypeStruct((R, C, D), jnp.float32),
        scratch_types=[
            pltpu.VMEM((rows_per_w, C), jnp.int32),
            [pltpu.VMEM((C, D), jnp.float32) for _ in range(NBUF)],
            [pltpu.SemaphoreType.DMA for _ in range(NBUF)],
            [pltpu.SemaphoreType.DMA for _ in range(NBUF)],
        ],
        compiler_params=pltpu.CompilerParams(use_tc_tiling_on_sc=False),
    )
    def lookup(x_hbm, table_hbm, out_hbm, idx_v, rows, g_sems, w_sems):
        wid = lax.axis_index("s") * NC + lax.axis_index("c")
        base = wid * rows_per_w
        pltpu.sync_copy(x_hbm.at[pl.ds(base, rows_per_w)], idx_v)

        def gather_start(r, b):
            pltpu.async_copy(table_hbm.at[idx_v.at[r]], rows[b], g_sems[b])

        def gather_wait(b):
            pltpu.make_async_copy(
                table_hbm.at[idx_v.at[0]], rows[b], g_sems[b]
            ).wait()

        def writeback_start(r, b):
            pltpu.async_copy(rows[b], out_hbm.at[base + r], w_sems[b])

        def writeback_wait(b):
            pltpu.make_async_copy(rows[b], out_hbm.at[0], w_sems[b]).wait()

        for b in range(NBUF):
            gather_start(b, b)

        def body(g, carry):
            for b in range(NBUF):
                r = g + b
                gather_wait(b)
                writeback_start(r, b)

                @pl.when(r + NBUF < rows_per_w)
                def _():
                    writeback_wait(b)
                    gather_start(r + NBUF, b)

            return carry

        lax.fori_loop(0, rows_per_w // NBUF, lambda t, c: body(t * NBUF, c),
                      0, unroll=False)

        for b in range(NBUF):
            writeback_wait(b)

    return lookup


def kernel(x, table):
    R, C = x.shape
    V, D = table.shape
    return _make_lookup(R, C, D, NBUF=4)(x.astype(jnp.int32), table)
